# baseline (speedup 1.0000x reference)
import functools

import jax
import jax.numpy as jnp
import numpy as np
from jax import lax
from jax.experimental import pallas as pl
from jax.experimental.pallas import tpu as pltpu
from jax.experimental.pallas import tpu_sc as plsc

_N, _E, _D, _K = 10000, 320000, 128, 32
_NC, _NS = 2, 16
_NW = _NC * _NS
_NPAD = 10240
_NPW = _NPAD // _NW
_PAIR = 2 * _NPW
_S0 = 160
_NPWMAX = max(_S0, _PAIR - _S0)
_B = 4
_NB = _NPW // _B
_RB = _B * _K
_NBUF = 2
_G = _D // 32

_COL_PERM = np.empty((_D,), dtype=np.int32)
for _g in range(_G):
    for _t in range(16):
        _COL_PERM[_g * 32 + 2 * _t] = _g * 32 + _t
        _COL_PERM[_g * 32 + 2 * _t + 1] = _g * 32 + 16 + _t


def _support_matmul(x, w, b2d):
    def body(x_ref, w_ref, b_ref, o_ref):
        o_ref[...] = (lax.dot_general(
            x_ref[...], w_ref[...], (((1,), (1,)), ((), ())),
            preferred_element_type=jnp.float32)
            + b_ref[...]).astype(jnp.bfloat16)

    return pl.pallas_call(
        body,
        grid=(10,),
        in_specs=[
            pl.BlockSpec((1000, _D), lambda i: (i, 0)),
            pl.BlockSpec((_D, _D), lambda i: (0, 0)),
            pl.BlockSpec((1, _D), lambda i: (0, 0)),
        ],
        out_specs=pl.BlockSpec((1000, _D), lambda i: (i, 0)),
        out_shape=jax.ShapeDtypeStruct((_N, _D), jnp.bfloat16),
    )(x, w, b2d)


def _make_sc_kernel():
    mesh = plsc.VectorSubcoreMesh(core_axis_name="c", subcore_axis_name="s")

    @functools.partial(
        pl.kernel,
        out_type=jax.ShapeDtypeStruct((_NPAD, _D), jnp.float32),
        mesh=mesh,
        compiler_params=pltpu.CompilerParams(use_tc_tiling_on_sc=False),
        scratch_types=[
            pltpu.VMEM((_NPWMAX * _K,), jnp.int32),
            pltpu.VMEM((_NPWMAX * _K,), jnp.int32),
            pltpu.VMEM((_NBUF, _RB, _D // 2), jnp.int32),
            pltpu.VMEM((_NBUF, _RB, _D), jnp.float32),
            pltpu.VMEM((_NBUF, _B, _D), jnp.float32),
            pltpu.VMEM((_D,), jnp.float32),
            pltpu.VMEM_SHARED((_N, _D // 2), jnp.int32),
            pltpu.VMEM((52, _D // 2), jnp.int32),
            [pltpu.SemaphoreType.DMA] * _NBUF,
            [pltpu.SemaphoreType.DMA] * _NBUF,
        ],
    )
    def sc(support_hbm, fedges_hbm, idxa_hbm, idxf_hbm, bias_hbm, out_hbm,
           idxa_v, idxf_v, arows, frows, outb, bias_v, sup_sh, bounce,
           sems_a, sems_f):
        cid = lax.axis_index("c")
        sid = lax.axis_index("s")
        base = sid * _PAIR + cid * _S0
        npw = jnp.where(cid == 0, _S0, _PAIR - _S0)
        nb = npw // _B

        def stage_chunk(row0, nrows):
            pltpu.sync_copy(support_hbm.at[pl.ds(row0, nrows)],
                            bounce.at[pl.ds(0, nrows)])
            pltpu.sync_copy(bounce.at[pl.ds(0, nrows)],
                            sup_sh.at[pl.ds(row0, nrows)])

        for _ch in range(12):
            stage_chunk(sid * 624 + _ch * 52, 52)

        @pl.when(sid == _NS - 1)
        def _():
            stage_chunk(15 * 624 + 624, 16)

        pltpu.sync_copy(idxa_hbm.at[pl.ds(base * _K, _NPWMAX * _K)], idxa_v)
        pltpu.sync_copy(idxf_hbm.at[pl.ds(base * _K, _NPWMAX * _K)], idxf_v)
        pltpu.sync_copy(bias_hbm, bias_v)
        plsc.subcore_barrier()

        def copies(j, slot):
            a_cp = pltpu.make_async_copy(
                sup_sh.at[idxa_v.at[pl.ds(j * _RB, _RB)]],
                arows.at[slot], sems_a[slot])
            f_cp = pltpu.make_async_copy(
                fedges_hbm.at[idxf_v.at[pl.ds(j * _RB, _RB)]],
                frows.at[slot], sems_f[slot])
            return a_cp, f_cp

        def fire(j, slot):
            a_cp, f_cp = copies(j, slot)
            a_cp.start()
            f_cp.start()

        def consume(j, slot):
            a_cp, f_cp = copies(j, slot)
            a_cp.wait()
            f_cp.wait()
            for n in range(_B):
                def group_body(g, carry, n=n):
                    off = g * 32
                    pa = [None] * 2
                    pb = [None] * 2
                    for k in range(_K):
                        r = n * _K + k
                        ab = arows[slot, r, pl.ds(g * 16, 16)]
                        a_lo = lax.bitcast_convert_type(
                            lax.shift_left(ab, 16), jnp.float32)
                        a_hi = lax.bitcast_convert_type(
                            lax.bitwise_and(ab, jnp.int32(-65536)),
                            jnp.float32)
                        f_lo = frows[slot, r, pl.ds(off, 16)]
                        f_hi = frows[slot, r, pl.ds(off + 16, 16)]
                        prod_a = a_lo * f_lo
                        prod_b = a_hi * f_hi
                        p = k % 2
                        pa[p] = prod_a if pa[p] is None else pa[p] + prod_a
                        pb[p] = prod_b if pb[p] is None else pb[p] + prod_b
                    for acc, sub in ((pa[0] + pa[1], 0), (pb[0] + pb[1], 16)):
                        e = jnp.exp(acc * 2.0)
                        t = 1.0 - 2.0 / (e + 1.0)
                        outb[slot, n, pl.ds(off + sub, 16)] = (
                            t + bias_v[pl.ds(off + sub, 16)])
                    return carry

                lax.fori_loop(0, _G, group_body, 0)
            pltpu.sync_copy(outb.at[slot], out_hbm.at[pl.ds(base + j * _B, _B)])

        for s in range(_NBUF):
            fire(s, s)

        def step(jj, carry):
            j = jj * _NBUF
            for slot in range(_NBUF):
                consume(j + slot, slot)

                @pl.when(j + slot + _NBUF < nb)
                def _():
                    fire(j + slot + _NBUF, slot)

            return carry

        lax.fori_loop(0, nb // _NBUF, step, 0)

    return sc


_sc_kernel = _make_sc_kernel()


def kernel(input_features, f_nodes, f_bonds, node2edge, edge2node, b2revb,
           fedges, a2a, W, b_lin, bias):
    perm = jnp.asarray(_COL_PERM)
    support = _support_matmul(input_features, W[perm, :],
                              b_lin[perm].reshape(1, _D))
    support = lax.bitcast_convert_type(
        support.reshape(_N, _D // 2, 2), jnp.int32)
    pad = _NPAD - _N
    idxa = jnp.concatenate(
        [a2a, jnp.zeros((pad, _K), jnp.int32)], axis=0).reshape(-1)
    idxf = jnp.concatenate(
        [node2edge, jnp.zeros((pad, _K), jnp.int32)], axis=0).reshape(-1)
    out = _sc_kernel(support, fedges, idxa, idxf, bias)
    return out[:_N]


# speedup vs baseline: 1.2675x; 1.2675x over previous
import functools

import jax
import jax.numpy as jnp
import numpy as np
from jax import lax
from jax.experimental import pallas as pl
from jax.experimental.pallas import tpu as pltpu
from jax.experimental.pallas import tpu_sc as plsc

_N, _E, _D, _K = 10000, 320000, 128, 32
_NC, _NS = 2, 16
_NW = _NC * _NS
_NPAD = 10240
_NPW = _NPAD // _NW
_PAIR = 2 * _NPW
_S0 = 480
_NPWMAX = max(_S0, _PAIR - _S0)
_B = 4
_NB = _NPW // _B
_RB = _B * _K
_NBUF = 2
_G = _D // 32

_COL_PERM = np.empty((_D,), dtype=np.int32)
for _g in range(_G):
    for _t in range(16):
        _COL_PERM[_g * 32 + 2 * _t] = _g * 32 + _t
        _COL_PERM[_g * 32 + 2 * _t + 1] = _g * 32 + 16 + _t


def _support_matmul(x, w, b2d):
    def body(x_ref, w_ref, b_ref, o_ref):
        o_ref[...] = (lax.dot_general(
            x_ref[...], w_ref[...], (((1,), (1,)), ((), ())),
            preferred_element_type=jnp.float32)
            + b_ref[...]).astype(jnp.bfloat16)

    return pl.pallas_call(
        body,
        grid=(10,),
        in_specs=[
            pl.BlockSpec((1000, _D), lambda i: (i, 0)),
            pl.BlockSpec((_D, _D), lambda i: (0, 0)),
            pl.BlockSpec((1, _D), lambda i: (0, 0)),
        ],
        out_specs=pl.BlockSpec((1000, _D), lambda i: (i, 0)),
        out_shape=jax.ShapeDtypeStruct((_N, _D), jnp.bfloat16),
    )(x, w, b2d)


def _make_sc_kernel():
    mesh = plsc.VectorSubcoreMesh(core_axis_name="c", subcore_axis_name="s")

    @functools.partial(
        pl.kernel,
        out_type=jax.ShapeDtypeStruct((_NPAD, _D), jnp.float32),
        mesh=mesh,
        compiler_params=pltpu.CompilerParams(use_tc_tiling_on_sc=False),
        scratch_types=[
            pltpu.VMEM((_NPWMAX * _K,), jnp.int32),
            pltpu.VMEM((_NPWMAX * _K,), jnp.int32),
            pltpu.VMEM((_NBUF, _RB, _D // 2), jnp.int32),
            pltpu.VMEM((_NBUF, _RB, _D), jnp.float32),
            pltpu.VMEM((_NBUF, _B, _D), jnp.float32),
            pltpu.VMEM((_D,), jnp.float32),
            pltpu.VMEM_SHARED((_N, _D // 2), jnp.int32),
            pltpu.VMEM((52, _D // 2), jnp.int32),
            [pltpu.SemaphoreType.DMA] * _NBUF,
            [pltpu.SemaphoreType.DMA] * _NBUF,
        ],
    )
    def sc(support_hbm, fedges_hbm, idxa_hbm, idxf_hbm, bias_hbm, out_hbm,
           idxa_v, idxf_v, arows, frows, outb, bias_v, sup_sh, bounce,
           sems_a, sems_f):
        cid = lax.axis_index("c")
        sid = lax.axis_index("s")
        base = sid * _PAIR + cid * _S0
        npw = jnp.where(cid == 0, _S0, _PAIR - _S0)
        nb = npw // _B

        def stage_chunk(row0, nrows):
            pltpu.sync_copy(support_hbm.at[pl.ds(row0, nrows)],
                            bounce.at[pl.ds(0, nrows)])
            pltpu.sync_copy(bounce.at[pl.ds(0, nrows)],
                            sup_sh.at[pl.ds(row0, nrows)])

        for _ch in range(12):
            stage_chunk(sid * 624 + _ch * 52, 52)

        @pl.when(sid == _NS - 1)
        def _():
            stage_chunk(15 * 624 + 624, 16)

        pltpu.sync_copy(idxa_hbm.at[pl.ds(base * _K, _NPWMAX * _K)], idxa_v)
        pltpu.sync_copy(idxf_hbm.at[pl.ds(base * _K, _NPWMAX * _K)], idxf_v)
        pltpu.sync_copy(bias_hbm, bias_v)
        plsc.subcore_barrier()

        def copies(j, slot):
            a_cp = pltpu.make_async_copy(
                sup_sh.at[idxa_v.at[pl.ds(j * _RB, _RB)]],
                arows.at[slot], sems_a[slot])
            f_cp = pltpu.make_async_copy(
                fedges_hbm.at[idxf_v.at[pl.ds(j * _RB, _RB)]],
                frows.at[slot], sems_f[slot])
            return a_cp, f_cp

        def fire(j, slot):
            a_cp, f_cp = copies(j, slot)
            a_cp.start()
            f_cp.start()

        def consume(j, slot):
            a_cp, f_cp = copies(j, slot)
            a_cp.wait()
            f_cp.wait()
            for n in range(_B):
                def group_body(g, carry, n=n):
                    off = g * 32
                    pa = [None] * 2
                    pb = [None] * 2
                    for k in range(_K):
                        r = n * _K + k
                        ab = arows[slot, r, pl.ds(g * 16, 16)]
                        a_lo = lax.bitcast_convert_type(
                            lax.shift_left(ab, 16), jnp.float32)
                        a_hi = lax.bitcast_convert_type(
                            lax.bitwise_and(ab, jnp.int32(-65536)),
                            jnp.float32)
                        f_lo = frows[slot, r, pl.ds(off, 16)]
                        f_hi = frows[slot, r, pl.ds(off + 16, 16)]
                        prod_a = a_lo * f_lo
                        prod_b = a_hi * f_hi
                        p = k % 2
                        pa[p] = prod_a if pa[p] is None else pa[p] + prod_a
                        pb[p] = prod_b if pb[p] is None else pb[p] + prod_b
                    for acc, sub in ((pa[0] + pa[1], 0), (pb[0] + pb[1], 16)):
                        e = jnp.exp(acc * 2.0)
                        t = 1.0 - 2.0 / (e + 1.0)
                        outb[slot, n, pl.ds(off + sub, 16)] = (
                            t + bias_v[pl.ds(off + sub, 16)])
                    return carry

                lax.fori_loop(0, _G, group_body, 0)
            pltpu.sync_copy(outb.at[slot], out_hbm.at[pl.ds(base + j * _B, _B)])

        for s in range(_NBUF):
            fire(s, s)

        def step(jj, carry):
            j = jj * _NBUF
            for slot in range(_NBUF):
                consume(j + slot, slot)

                @pl.when(j + slot + _NBUF < nb)
                def _():
                    fire(j + slot + _NBUF, slot)

            return carry

        lax.fori_loop(0, nb // _NBUF, step, 0)

    return sc


_sc_kernel = _make_sc_kernel()


def kernel(input_features, f_nodes, f_bonds, node2edge, edge2node, b2revb,
           fedges, a2a, W, b_lin, bias):
    perm = jnp.asarray(_COL_PERM)
    support = _support_matmul(input_features, W[perm, :],
                              b_lin[perm].reshape(1, _D))
    support = lax.bitcast_convert_type(
        support.reshape(_N, _D // 2, 2), jnp.int32)
    pad = _NPAD - _N
    idxa = jnp.concatenate(
        [a2a, jnp.zeros((pad, _K), jnp.int32)], axis=0).reshape(-1)
    idxf = jnp.concatenate(
        [node2edge, jnp.zeros((pad, _K), jnp.int32)], axis=0).reshape(-1)
    out = _sc_kernel(support, fedges, idxa, idxf, bias)
    return out[:_N]


# speedup vs baseline: 1.2684x; 1.0007x over previous
import functools

import jax
import jax.numpy as jnp
import numpy as np
from jax import lax
from jax.experimental import pallas as pl
from jax.experimental.pallas import tpu as pltpu
from jax.experimental.pallas import tpu_sc as plsc

_N, _E, _D, _K = 10000, 320000, 128, 32
_NC, _NS = 2, 16
_NW = _NC * _NS
_NPAD = 10240
_NPW = _NPAD // _NW
_PAIR = 2 * _NPW
_S0 = 480
_NPWMAX = max(_S0, _PAIR - _S0)
_B = 4
_NB = _NPW // _B
_RB = _B * _K
_NBUF = 2
_G = _D // 32

_COL_PERM = np.empty((_D,), dtype=np.int32)
for _g in range(_G):
    for _t in range(16):
        _COL_PERM[_g * 32 + 2 * _t] = _g * 32 + _t
        _COL_PERM[_g * 32 + 2 * _t + 1] = _g * 32 + 16 + _t


def _support_matmul(x, w, b2d):
    def body(x_ref, w_ref, b_ref, o_ref):
        o_ref[...] = (lax.dot_general(
            x_ref[...], w_ref[...], (((1,), (1,)), ((), ())),
            preferred_element_type=jnp.float32)
            + b_ref[...]).astype(jnp.bfloat16)

    return pl.pallas_call(
        body,
        grid=(10,),
        in_specs=[
            pl.BlockSpec((1000, _D), lambda i: (i, 0)),
            pl.BlockSpec((_D, _D), lambda i: (0, 0)),
            pl.BlockSpec((1, _D), lambda i: (0, 0)),
        ],
        out_specs=pl.BlockSpec((1000, _D), lambda i: (i, 0)),
        out_shape=jax.ShapeDtypeStruct((_N, _D), jnp.bfloat16),
    )(x, w, b2d)


def _make_sc_kernel():
    mesh = plsc.VectorSubcoreMesh(core_axis_name="c", subcore_axis_name="s")

    @functools.partial(
        pl.kernel,
        out_type=jax.ShapeDtypeStruct((_NPAD, _D), jnp.float32),
        mesh=mesh,
        compiler_params=pltpu.CompilerParams(use_tc_tiling_on_sc=False),
        scratch_types=[
            pltpu.VMEM((_NPWMAX * _K,), jnp.int32),
            pltpu.VMEM((_NPWMAX * _K,), jnp.int32),
            pltpu.VMEM((_NBUF, _RB, _D // 2), jnp.int32),
            pltpu.VMEM((_NBUF, _RB, _D), jnp.float32),
            pltpu.VMEM((_NBUF, _B, _D), jnp.float32),
            pltpu.VMEM((_D,), jnp.float32),
            pltpu.VMEM_SHARED((_N, _D // 2), jnp.int32),
            pltpu.VMEM((52, _D // 2), jnp.int32),
            [pltpu.SemaphoreType.DMA] * _NBUF,
            [pltpu.SemaphoreType.DMA] * _NBUF,
        ],
    )
    def sc(support_hbm, fedges_hbm, idxa_hbm, idxf_hbm, bias_hbm, out_hbm,
           idxa_v, idxf_v, arows, frows, outb, bias_v, sup_sh, bounce,
           sems_a, sems_f):
        cid = lax.axis_index("c")
        sid = lax.axis_index("s")
        base = sid * _PAIR + cid * _S0
        npw = jnp.where(cid == 0, _S0, _PAIR - _S0)
        nb = npw // _B

        def stage_chunk(row0, nrows):
            pltpu.sync_copy(support_hbm.at[pl.ds(row0, nrows)],
                            bounce.at[pl.ds(0, nrows)])
            pltpu.sync_copy(bounce.at[pl.ds(0, nrows)],
                            sup_sh.at[pl.ds(row0, nrows)])

        for _ch in range(12):
            stage_chunk(sid * 624 + _ch * 52, 52)

        @pl.when(sid == _NS - 1)
        def _():
            stage_chunk(15 * 624 + 624, 16)

        @pl.when(cid == 0)
        def _():
            pltpu.sync_copy(idxa_hbm.at[pl.ds(base * _K, _S0 * _K)],
                            idxa_v.at[pl.ds(0, _S0 * _K)])
            pltpu.sync_copy(idxf_hbm.at[pl.ds(base * _K, _S0 * _K)],
                            idxf_v.at[pl.ds(0, _S0 * _K)])

        @pl.when(cid == 1)
        def _():
            pltpu.sync_copy(
                idxa_hbm.at[pl.ds(base * _K, (_PAIR - _S0) * _K)],
                idxa_v.at[pl.ds(0, (_PAIR - _S0) * _K)])
            pltpu.sync_copy(
                idxf_hbm.at[pl.ds(base * _K, (_PAIR - _S0) * _K)],
                idxf_v.at[pl.ds(0, (_PAIR - _S0) * _K)])
        pltpu.sync_copy(bias_hbm, bias_v)
        plsc.subcore_barrier()

        def copies(j, slot):
            a_cp = pltpu.make_async_copy(
                sup_sh.at[idxa_v.at[pl.ds(j * _RB, _RB)]],
                arows.at[slot], sems_a[slot])
            f_cp = pltpu.make_async_copy(
                fedges_hbm.at[idxf_v.at[pl.ds(j * _RB, _RB)]],
                frows.at[slot], sems_f[slot])
            return a_cp, f_cp

        def fire(j, slot):
            a_cp, f_cp = copies(j, slot)
            a_cp.start()
            f_cp.start()

        def consume(j, slot):
            a_cp, f_cp = copies(j, slot)
            a_cp.wait()
            f_cp.wait()
            for n in range(_B):
                def group_body(g, carry, n=n):
                    off = g * 32
                    pa = [None] * 2
                    pb = [None] * 2
                    for k in range(_K):
                        r = n * _K + k
                        ab = arows[slot, r, pl.ds(g * 16, 16)]
                        a_lo = lax.bitcast_convert_type(
                            lax.shift_left(ab, 16), jnp.float32)
                        a_hi = lax.bitcast_convert_type(
                            lax.bitwise_and(ab, jnp.int32(-65536)),
                            jnp.float32)
                        f_lo = frows[slot, r, pl.ds(off, 16)]
                        f_hi = frows[slot, r, pl.ds(off + 16, 16)]
                        prod_a = a_lo * f_lo
                        prod_b = a_hi * f_hi
                        p = k % 2
                        pa[p] = prod_a if pa[p] is None else pa[p] + prod_a
                        pb[p] = prod_b if pb[p] is None else pb[p] + prod_b
                    for acc, sub in ((pa[0] + pa[1], 0), (pb[0] + pb[1], 16)):
                        e = jnp.exp(acc * 2.0)
                        t = 1.0 - 2.0 / (e + 1.0)
                        outb[slot, n, pl.ds(off + sub, 16)] = (
                            t + bias_v[pl.ds(off + sub, 16)])
                    return carry

                lax.fori_loop(0, _G, group_body, 0)
            pltpu.sync_copy(outb.at[slot], out_hbm.at[pl.ds(base + j * _B, _B)])

        for s in range(_NBUF):
            fire(s, s)

        def step(jj, carry):
            j = jj * _NBUF
            for slot in range(_NBUF):
                consume(j + slot, slot)

                @pl.when(j + slot + _NBUF < nb)
                def _():
                    fire(j + slot + _NBUF, slot)

            return carry

        lax.fori_loop(0, nb // _NBUF, step, 0)

    return sc


_sc_kernel = _make_sc_kernel()


def kernel(input_features, f_nodes, f_bonds, node2edge, edge2node, b2revb,
           fedges, a2a, W, b_lin, bias):
    perm = jnp.asarray(_COL_PERM)
    support = _support_matmul(input_features, W[perm, :],
                              b_lin[perm].reshape(1, _D))
    support = lax.bitcast_convert_type(
        support.reshape(_N, _D // 2, 2), jnp.int32)
    pad = _NPAD - _N
    idxa = jnp.concatenate(
        [a2a, jnp.zeros((pad, _K), jnp.int32)], axis=0).reshape(-1)
    idxf = jnp.concatenate(
        [node2edge, jnp.zeros((pad, _K), jnp.int32)], axis=0).reshape(-1)
    out = _sc_kernel(support, fedges, idxa, idxf, bias)
    return out[:_N]
